# baseline (speedup 1.0000x reference)
import jax
import jax.numpy as jnp
from jax import lax
from jax.experimental import pallas as pl
from jax.experimental.pallas import tpu as pltpu
from jax.experimental.pallas import tpu_sc as plsc

C = 1203
D = 256
N = 65536
TEMP = 0.07
CP = 1280
NC, NS = 2, 16
NW = NC * NS
ND = 4
DSUB = D // ND
NR = NW // ND
ROWS_R = N // NR
CHUNK = 32
NCH = ROWS_R // CHUNK
NBUF = 8
NEG = -1e30


def _sc_body(feats_hbm, labels_hbm, sums_hbm, cnt_hbm,
             labels_v, f0, f1, f2, f3, f4, f5, f6, f7, acc_v, cnt_v,
             sem0, sem1, sem2, sem3, sem4, sem5, sem6, sem7):
    cid = lax.axis_index("c")
    sid = lax.axis_index("s")
    wid = cid * NS + sid
    r = wid // ND
    d = wid % ND
    row0 = r * ROWS_R
    col0 = d * DSUB

    zero16 = jnp.zeros((16,), jnp.float32)
    ones16 = jnp.ones((16,), jnp.float32)
    lane = lax.iota(jnp.int32, 16)
    cols = [lane + k * 16 for k in range(DSUB // 16)]


    def dma_start(j, buf, sem):
        pltpu.async_copy(
            feats_hbm.at[pl.ds(row0 + j * CHUNK, CHUNK), pl.ds(col0, DSUB)],
            buf, sem)

    def dma_wait(buf, sem):
        pltpu.make_async_copy(
            feats_hbm.at[pl.ds(row0, CHUNK), pl.ds(col0, DSUB)],
            buf, sem).wait()

    def compute(j, buf):
        jbase = j * CHUNK

        @pl.when(d == 0)
        def _():
            @plsc.parallel_loop(0, CHUNK // 16, 1, unroll=2)
            def _cnt16(t):
                lv = labels_v[pl.ds(jbase + t * 16, 16)]
                plsc.addupdate_scatter(cnt_v, [lv, lane], ones16)

        @plsc.parallel_loop(0, CHUNK, 1, unroll=8)
        def _rows(i):
            ridx = jnp.full((16,), jbase + i, jnp.int32)
            bl = plsc.load_gather(labels_v, [ridx])
            for k in range(DSUB // 16):
                v = buf[i, pl.ds(k * 16, 16)]
                plsc.addupdate_scatter(acc_v, [bl, cols[k]], v)

    bufs = (f0, f1, f2, f3, f4, f5, f6, f7)
    sems = (sem0, sem1, sem2, sem3, sem4, sem5, sem6, sem7)
    for b in range(NBUF - 1):
        dma_start(b, bufs[b], sems[b])
    pltpu.sync_copy(labels_hbm.at[pl.ds(row0, ROWS_R)], labels_v)

    @plsc.parallel_loop(0, CP, 1, unroll=4)
    def _zero(i):
        for k in range(DSUB // 16):
            acc_v[i, pl.ds(k * 16, 16)] = zero16
        cnt_v[i, :] = zero16

    def outer(jj, c):
        j = jj * NBUF
        for b in range(NBUF):
            dma_wait(bufs[b], sems[b])
            nb = (b + NBUF - 1) % NBUF
            dma_start(jnp.minimum(j + b + NBUF - 1, NCH - 1), bufs[nb], sems[nb])
            compute(j + b, bufs[b])
        return c
    lax.fori_loop(0, NCH // NBUF, outer, 0)
    for b in range(NBUF - 1):
        dma_wait(bufs[b], sems[b])

    pltpu.sync_copy(acc_v, sums_hbm.at[r, :, pl.ds(col0, DSUB)])

    @pl.when(d == 0)
    def _():
        pltpu.sync_copy(cnt_v, cnt_hbm.at[r])


def _segment_sums(cls_feats, labels):
    mesh = plsc.VectorSubcoreMesh(core_axis_name="c", subcore_axis_name="s",
                                  num_cores=NC, num_subcores=NS)
    return pl.kernel(
        _sc_body,
        out_type=(jax.ShapeDtypeStruct((NR, CP, D), jnp.float32),
                  jax.ShapeDtypeStruct((NR, CP, 16), jnp.float32)),
        mesh=mesh,
        compiler_params=pltpu.CompilerParams(use_tc_tiling_on_sc=False,
                                             needs_layout_passes=False),
        scratch_types=[
            pltpu.VMEM((ROWS_R,), jnp.int32),
            pltpu.VMEM((CHUNK, DSUB), jnp.float32),
            pltpu.VMEM((CHUNK, DSUB), jnp.float32),
            pltpu.VMEM((CHUNK, DSUB), jnp.float32),
            pltpu.VMEM((CHUNK, DSUB), jnp.float32),
            pltpu.VMEM((CHUNK, DSUB), jnp.float32),
            pltpu.VMEM((CHUNK, DSUB), jnp.float32),
            pltpu.VMEM((CHUNK, DSUB), jnp.float32),
            pltpu.VMEM((CHUNK, DSUB), jnp.float32),
            pltpu.VMEM((CP, DSUB), jnp.float32),
            pltpu.VMEM((CP, 16), jnp.float32),
            pltpu.SemaphoreType.DMA,
            pltpu.SemaphoreType.DMA,
            pltpu.SemaphoreType.DMA,
            pltpu.SemaphoreType.DMA,
            pltpu.SemaphoreType.DMA,
            pltpu.SemaphoreType.DMA,
            pltpu.SemaphoreType.DMA,
            pltpu.SemaphoreType.DMA,
        ],
    )(cls_feats, labels)


def _tc_loss(sums_ref, cnt_ref, prot_ref, dp_ref, out_ref):
    sums = sums_ref[0]
    for i in range(1, NR):
        sums = sums + sums_ref[i]
    c16 = cnt_ref[0]
    for i in range(1, NR):
        c16 = c16 + cnt_ref[i]
    counts = jnp.sum(c16, axis=1, keepdims=True)
    present = counts > 0.0
    means = sums / jnp.maximum(counts, 1.0)
    delta = jnp.where(present, means, dp_ref[...])
    prot = prot_ref[...]
    an = prot / (jnp.sqrt(jnp.sum(prot * prot, axis=1, keepdims=True)) + 1e-8)
    bn = delta / (jnp.sqrt(jnp.sum(delta * delta, axis=1, keepdims=True)) + 1e-8)
    logits = lax.dot_general(an, bn, (((1,), (1,)), ((), ())),
                             preferred_element_type=jnp.float32) / TEMP
    col = lax.broadcasted_iota(jnp.int32, (CP, CP), 1)
    logits = jnp.where(col < C, logits, NEG)
    m = jnp.max(logits, axis=1, keepdims=True)
    lse = m + jnp.log(jnp.sum(jnp.exp(logits - m), axis=1, keepdims=True))
    row = lax.broadcasted_iota(jnp.int32, (CP, CP), 0)
    diag = jnp.sum(jnp.where(row == col, logits, 0.0), axis=1, keepdims=True)
    per_row = lse - diag
    pf = jnp.where(present, 1.0, 0.0)
    num = jnp.sum(per_row * pf, axis=(0, 1), keepdims=True)
    den = jnp.maximum(jnp.sum(pf, axis=(0, 1), keepdims=True), 1.0)
    out_ref[...] = num / den


def kernel(cls_feats, cls_targets, prototypes, delta_prototype):
    labels = cls_targets.reshape(N).astype(jnp.int32)
    sums8, cnt8 = _segment_sums(cls_feats, labels)
    prot_pad = jnp.pad(prototypes, ((0, CP - C), (0, 0)))
    dp_pad = jnp.pad(delta_prototype, ((0, CP - C), (0, 0)))
    loss = pl.pallas_call(
        _tc_loss,
        out_shape=jax.ShapeDtypeStruct((1, 1), jnp.float32),
    )(sums8, cnt8, prot_pad, dp_pad)
    return loss[0, 0]


# speedup vs baseline: 1.0189x; 1.0189x over previous
import jax
import jax.numpy as jnp
from jax import lax
from jax.experimental import pallas as pl
from jax.experimental.pallas import tpu as pltpu
from jax.experimental.pallas import tpu_sc as plsc

C = 1203
D = 256
N = 65536
TEMP = 0.07
CP = 1280
NC, NS = 2, 16
NW = NC * NS
ND = 4
DSUB = D // ND
NR = NW // ND
ROWS_R = N // NR
CHUNK = 64
NCH = ROWS_R // CHUNK
NBUF = 4
NEG = -1e30


def _sc_body(feats_hbm, labels_hbm, sums_hbm, cnt_hbm,
             labels_v, f0, f1, f2, f3, acc_v, cnt_v,
             sem0, sem1, sem2, sem3):
    cid = lax.axis_index("c")
    sid = lax.axis_index("s")
    wid = cid * NS + sid
    r = wid // ND
    d = wid % ND
    row0 = r * ROWS_R
    col0 = d * DSUB

    zero16 = jnp.zeros((16,), jnp.float32)
    ones16 = jnp.ones((16,), jnp.float32)
    lane = lax.iota(jnp.int32, 16)
    cols = [lane + k * 16 for k in range(DSUB // 16)]


    def dma_start(j, buf, sem):
        pltpu.async_copy(
            feats_hbm.at[pl.ds(row0 + j * CHUNK, CHUNK), pl.ds(col0, DSUB)],
            buf, sem)

    def dma_wait(buf, sem):
        pltpu.make_async_copy(
            feats_hbm.at[pl.ds(row0, CHUNK), pl.ds(col0, DSUB)],
            buf, sem).wait()

    def compute(j, buf):
        jbase = j * CHUNK

        @pl.when(d == 0)
        def _():
            @plsc.parallel_loop(0, CHUNK // 16, 1, unroll=2)
            def _cnt16(t):
                lv = labels_v[pl.ds(jbase + t * 16, 16)]
                plsc.addupdate_scatter(cnt_v, [lv, lane], ones16)

        @plsc.parallel_loop(0, CHUNK, 1, unroll=8)
        def _rows(i):
            ridx = jnp.full((16,), jbase + i, jnp.int32)
            bl = plsc.load_gather(labels_v, [ridx])
            for k in range(DSUB // 16):
                v = buf[i, pl.ds(k * 16, 16)]
                plsc.addupdate_scatter(acc_v, [bl, cols[k]], v)

    bufs = (f0, f1, f2, f3)
    sems = (sem0, sem1, sem2, sem3)
    for b in range(NBUF - 1):
        dma_start(b, bufs[b], sems[b])
    pltpu.sync_copy(labels_hbm.at[pl.ds(row0, ROWS_R)], labels_v)

    @plsc.parallel_loop(0, CP, 1, unroll=4)
    def _zero(i):
        for k in range(DSUB // 16):
            acc_v[i, pl.ds(k * 16, 16)] = zero16
        cnt_v[i, :] = zero16

    def outer(jj, c):
        j = jj * NBUF
        for b in range(NBUF):
            dma_wait(bufs[b], sems[b])
            nb = (b + NBUF - 1) % NBUF
            dma_start(jnp.minimum(j + b + NBUF - 1, NCH - 1), bufs[nb], sems[nb])
            compute(j + b, bufs[b])
        return c
    lax.fori_loop(0, NCH // NBUF, outer, 0)
    for b in range(NBUF - 1):
        dma_wait(bufs[b], sems[b])

    pltpu.sync_copy(acc_v, sums_hbm.at[r, :, pl.ds(col0, DSUB)])

    @pl.when(d == 0)
    def _():
        pltpu.sync_copy(cnt_v, cnt_hbm.at[r])


def _segment_sums(cls_feats, labels):
    mesh = plsc.VectorSubcoreMesh(core_axis_name="c", subcore_axis_name="s",
                                  num_cores=NC, num_subcores=NS)
    return pl.kernel(
        _sc_body,
        out_type=(jax.ShapeDtypeStruct((NR, CP, D), jnp.float32),
                  jax.ShapeDtypeStruct((NR, CP, 16), jnp.float32)),
        mesh=mesh,
        compiler_params=pltpu.CompilerParams(use_tc_tiling_on_sc=False,
                                             needs_layout_passes=False),
        scratch_types=[
            pltpu.VMEM((ROWS_R,), jnp.int32),
            pltpu.VMEM((CHUNK, DSUB), jnp.float32),
            pltpu.VMEM((CHUNK, DSUB), jnp.float32),
            pltpu.VMEM((CHUNK, DSUB), jnp.float32),
            pltpu.VMEM((CHUNK, DSUB), jnp.float32),
            pltpu.VMEM((CP, DSUB), jnp.float32),
            pltpu.VMEM((CP, 16), jnp.float32),
            pltpu.SemaphoreType.DMA,
            pltpu.SemaphoreType.DMA,
            pltpu.SemaphoreType.DMA,
            pltpu.SemaphoreType.DMA,
        ],
    )(cls_feats, labels)


def _tc_loss(sums_ref, cnt_ref, prot_ref, dp_ref, out_ref):
    sums = sums_ref[0]
    for i in range(1, NR):
        sums = sums + sums_ref[i]
    c16 = cnt_ref[0]
    for i in range(1, NR):
        c16 = c16 + cnt_ref[i]
    counts = jnp.sum(c16, axis=1, keepdims=True)
    present = counts > 0.0
    means = sums / jnp.maximum(counts, 1.0)
    delta = jnp.where(present, means, dp_ref[...])
    prot = prot_ref[...]
    an = prot / (jnp.sqrt(jnp.sum(prot * prot, axis=1, keepdims=True)) + 1e-8)
    bn = delta / (jnp.sqrt(jnp.sum(delta * delta, axis=1, keepdims=True)) + 1e-8)
    logits = lax.dot_general(an, bn, (((1,), (1,)), ((), ())),
                             preferred_element_type=jnp.float32) / TEMP
    col = lax.broadcasted_iota(jnp.int32, (CP, CP), 1)
    logits = jnp.where(col < C, logits, NEG)
    m = jnp.max(logits, axis=1, keepdims=True)
    lse = m + jnp.log(jnp.sum(jnp.exp(logits - m), axis=1, keepdims=True))
    row = lax.broadcasted_iota(jnp.int32, (CP, CP), 0)
    diag = jnp.sum(jnp.where(row == col, logits, 0.0), axis=1, keepdims=True)
    per_row = lse - diag
    pf = jnp.where(present, 1.0, 0.0)
    num = jnp.sum(per_row * pf, axis=(0, 1), keepdims=True)
    den = jnp.maximum(jnp.sum(pf, axis=(0, 1), keepdims=True), 1.0)
    out_ref[...] = num / den


def kernel(cls_feats, cls_targets, prototypes, delta_prototype):
    labels = cls_targets.reshape(N).astype(jnp.int32)
    sums8, cnt8 = _segment_sums(cls_feats, labels)
    prot_pad = jnp.pad(prototypes, ((0, CP - C), (0, 0)))
    dp_pad = jnp.pad(delta_prototype, ((0, CP - C), (0, 0)))
    loss = pl.pallas_call(
        _tc_loss,
        out_shape=jax.ShapeDtypeStruct((1, 1), jnp.float32),
    )(sums8, cnt8, prot_pad, dp_pad)
    return loss[0, 0]
